# Initial kernel scaffold; baseline (speedup 1.0000x reference)
import jax
import jax.numpy as jnp
from jax.experimental import pallas as pl


def kernel(unary, binary, index1, index2):
    raise NotImplementedError("write your pallas kernel here")


# speedup vs baseline: 2.7602x; 2.7602x over previous
import functools

import jax
import jax.numpy as jnp
from jax import lax
from jax.experimental import pallas as pl
from jax.experimental.pallas import tpu as pltpu
from jax.experimental.pallas import tpu_sc as plsc


def kernel(unary, binary, index1, index2):
    V, D = unary.shape
    B, F = binary.shape
    out_cols = 2 * D + F

    info = plsc.get_sparse_core_info()
    NC, NS = info.num_cores, info.num_subcores
    NW = NC * NS
    per_w = B // NW
    E = 200
    n_chunks = per_w // E

    mesh = plsc.VectorSubcoreMesh(core_axis_name="c", subcore_axis_name="s")

    @functools.partial(
        pl.kernel,
        mesh=mesh,
        out_type=jax.ShapeDtypeStruct((B, out_cols), jnp.float32),
        scratch_types=[
            pltpu.VMEM((per_w,), jnp.int32),
            pltpu.VMEM((per_w,), jnp.int32),
            pltpu.VMEM((E, D), jnp.float32),
            pltpu.VMEM((E, D), jnp.float32),
            pltpu.VMEM((E, F), jnp.float32),
            pltpu.SemaphoreType.DMA,
            pltpu.SemaphoreType.DMA,
        ],
    )
    def join_k(unary_hbm, binary_hbm, idx1_hbm, idx2_hbm, out_hbm,
               idx1_v, idx2_v, rows1_v, rows2_v, bin_v, sem1, sem2):
        wid = lax.axis_index("s") * NC + lax.axis_index("c")
        base = wid * per_w
        pltpu.sync_copy(idx1_hbm.at[pl.ds(base, per_w)], idx1_v)
        pltpu.sync_copy(idx2_hbm.at[pl.ds(base, per_w)], idx2_v)

        def body(i, carry):
            off = i * E
            g1 = pltpu.async_copy(
                unary_hbm.at[idx1_v.at[pl.ds(off, E)]], rows1_v, sem1)
            g2 = pltpu.async_copy(
                unary_hbm.at[idx2_v.at[pl.ds(off, E)]], rows2_v, sem2)
            pltpu.sync_copy(binary_hbm.at[pl.ds(base + off, E)], bin_v)
            pltpu.sync_copy(bin_v, out_hbm.at[pl.ds(base + off, E),
                                              pl.ds(2 * D, F)])
            g1.wait()
            pltpu.sync_copy(rows1_v, out_hbm.at[pl.ds(base + off, E),
                                                pl.ds(0, D)])
            g2.wait()
            pltpu.sync_copy(rows2_v, out_hbm.at[pl.ds(base + off, E),
                                                pl.ds(D, D)])
            return carry

        lax.fori_loop(0, n_chunks, body, 0)

    return join_k(unary, binary, index1, index2)
